# Initial kernel scaffold; baseline (speedup 1.0000x reference)
import jax
import jax.numpy as jnp
from jax.experimental import pallas as pl


def kernel(user_embedding, item_embedding, user_features, item_features, wr_w, wr_b, social_indices, social_values, ui_indices, ui_values, user_list, item_list):
    raise NotImplementedError("write your pallas kernel here")


# speedup vs baseline: 5.6932x; 5.6932x over previous
import functools

import jax
import jax.numpy as jnp
from jax import lax
from jax.experimental import pallas as pl
from jax.experimental.pallas import tpu as pltpu
from jax.experimental.pallas import tpu_sc as plsc

N_ROWS = 100000
N_FEAT = 128
N_DIM = 32
HALF = 16
N_EDGES = 1600000
N_BATCH = 16384

NC = 2
NS = 16
LANES = 16

CH = 1024
SUB = 128
NSUB = CH // SUB
EPW = 100352
NCHUNK = EPW // CH
E_PAD = EPW * NS
NRW = EPW // SUB
STRIPE = 6248
TAIL_OFF = STRIPE * NS
TAIL = N_ROWS - TAIL_OFF

BR = 1000
GRID = N_ROWS // BR

_f32 = jnp.float32
_i32 = jnp.int32


def _mm_stats_body(f_ref, w_ref, z_ref, s1_ref, s2_ref, wcs_ref):
    i = pl.program_id(0)
    f = f_ref[...]
    z_ref[...] = jnp.dot(f, w_ref[...], preferred_element_type=_f32)

    @pl.when(i == 0)
    def _():
        s1_ref[...] = jnp.zeros((1, 1), _f32)
        s2_ref[...] = jnp.zeros((1, 1), _f32)
        wcs_ref[...] = jnp.sum(w_ref[...], axis=0, keepdims=True)

    s1_ref[...] += jnp.sum(f).reshape(1, 1)
    s2_ref[...] += jnp.sum(f * f).reshape(1, 1)


def _mm_stats(features, w):
    return pl.pallas_call(
        _mm_stats_body,
        grid=(GRID,),
        in_specs=[
            pl.BlockSpec((BR, N_FEAT), lambda i: (i, 0)),
            pl.BlockSpec((N_FEAT, N_DIM), lambda i: (0, 0)),
        ],
        out_specs=[
            pl.BlockSpec((BR, N_DIM), lambda i: (i, 0)),
            pl.BlockSpec((1, 1), lambda i: (0, 0)),
            pl.BlockSpec((1, 1), lambda i: (0, 0)),
            pl.BlockSpec((1, N_DIM), lambda i: (0, 0)),
        ],
        out_shape=[
            jax.ShapeDtypeStruct((N_ROWS, N_DIM), _f32),
            jax.ShapeDtypeStruct((1, 1), _f32),
            jax.ShapeDtypeStruct((1, 1), _f32),
            jax.ShapeDtypeStruct((1, N_DIM), _f32),
        ],
    )(features, w)


def _sig_stats_body(z_ref, a_ref, c_ref, s_ref, s1_ref, s2_ref):
    i = pl.program_id(0)
    z = z_ref[...]
    s = 1.0 / (1.0 + jnp.exp(-(z * a_ref[...] + c_ref[...])))
    s_ref[...] = s

    @pl.when(i == 0)
    def _():
        s1_ref[...] = jnp.zeros((1, 1), _f32)
        s2_ref[...] = jnp.zeros((1, 1), _f32)

    s1_ref[...] += jnp.sum(s).reshape(1, 1)
    s2_ref[...] += jnp.sum(s * s).reshape(1, 1)


def _sig_stats(z, a, c):
    return pl.pallas_call(
        _sig_stats_body,
        grid=(GRID,),
        in_specs=[
            pl.BlockSpec((BR, N_DIM), lambda i: (i, 0)),
            pl.BlockSpec((1, 1), lambda i: (0, 0)),
            pl.BlockSpec((1, N_DIM), lambda i: (0, 0)),
        ],
        out_specs=[
            pl.BlockSpec((BR, N_DIM), lambda i: (i, 0)),
            pl.BlockSpec((1, 1), lambda i: (0, 0)),
            pl.BlockSpec((1, 1), lambda i: (0, 0)),
        ],
        out_shape=[
            jax.ShapeDtypeStruct((N_ROWS, N_DIM), _f32),
            jax.ShapeDtypeStruct((1, 1), _f32),
            jax.ShapeDtypeStruct((1, 1), _f32),
        ],
    )(z, a, c)


def _fusion_body(s_ref, emb_ref, m_ref, is_ref, lo_ref, hi_ref):
    fus = (s_ref[...] - m_ref[...]) * is_ref[...] + emb_ref[...]
    lo_ref[...] = fus[:, :HALF]
    hi_ref[...] = fus[:, HALF:]


def _fusion(s, emb, m, inv_s):
    return pl.pallas_call(
        _fusion_body,
        grid=(GRID,),
        in_specs=[
            pl.BlockSpec((BR, N_DIM), lambda i: (i, 0)),
            pl.BlockSpec((BR, N_DIM), lambda i: (i, 0)),
            pl.BlockSpec((1, 1), lambda i: (0, 0)),
            pl.BlockSpec((1, 1), lambda i: (0, 0)),
        ],
        out_specs=[
            pl.BlockSpec((BR, HALF), lambda i: (i, 0)),
            pl.BlockSpec((BR, HALF), lambda i: (i, 0)),
        ],
        out_shape=[
            jax.ShapeDtypeStruct((N_ROWS, HALF), _f32),
            jax.ShapeDtypeStruct((N_ROWS, HALF), _f32),
        ],
    )(s, emb, m, inv_s)


def _fusion_halves(features, w, b, emb):
    z, s1, s2, wcs = _mm_stats(features, w)
    n = float(N_ROWS * N_FEAT)
    m = s1 / n
    inv_s = 1.0 / jnp.sqrt((s2 - n * m * m) / (n - 1.0))
    c = b.reshape(1, N_DIM) - (m * inv_s) * wcs
    s, t1, t2 = _sig_stats(z, inv_s, c)
    n2 = float(N_ROWS * N_DIM)
    m2 = t1 / n2
    inv_s2 = 1.0 / jnp.sqrt((t2 - n2 * m2 * m2) / (n2 - 1.0))
    return _fusion(s, emb, m2, inv_s2)


@functools.cache
def _mesh():
    return plsc.VectorSubcoreMesh(
        core_axis_name="c", subcore_axis_name="s",
        num_cores=NC, num_subcores=NS)


def _spmm_body(dst_hbm, src_hbm, val_hbm, xlo_hbm, xhi_hbm, ilo_hbm, ihi_hbm,
               outlo_hbm, outhi_hbm, colv, rowv, valv, gath, acc, gsem, ssem):
    cid = lax.axis_index("c")
    sid = lax.axis_index("s")

    @pl.when(cid == 0)
    def _():
        pltpu.sync_copy(ilo_hbm.at[pl.ds(sid * STRIPE, STRIPE)],
                        acc.at[pl.ds(sid * STRIPE, STRIPE)])

        @pl.when(sid == 0)
        def _():
            pltpu.sync_copy(ilo_hbm.at[pl.ds(TAIL_OFF, TAIL)],
                            acc.at[pl.ds(TAIL_OFF, TAIL)])

    @pl.when(cid == 1)
    def _():
        pltpu.sync_copy(ihi_hbm.at[pl.ds(sid * STRIPE, STRIPE)],
                        acc.at[pl.ds(sid * STRIPE, STRIPE)])

        @pl.when(sid == 0)
        def _():
            pltpu.sync_copy(ihi_hbm.at[pl.ds(TAIL_OFF, TAIL)],
                            acc.at[pl.ds(TAIL_OFF, TAIL)])

    plsc.subcore_barrier()

    def chunk(ci, carry):
        r0 = sid * NRW + ci * NSUB
        pltpu.sync_copy(src_hbm.at[pl.ds(r0, NSUB)], colv)
        pltpu.sync_copy(dst_hbm.at[pl.ds(r0, NSUB)], rowv)
        pltpu.sync_copy(val_hbm.at[pl.ds(sid * EPW + ci * CH, CH)], valv)

        @pl.when(cid == 0)
        def _():
            ds = [pltpu.async_copy(xlo_hbm.at[colv.at[k]],
                                   gath.at[pl.ds(k * SUB, SUB)], gsem)
                  for k in range(NSUB)]
            for d in ds:
                d.wait()

        @pl.when(cid == 1)
        def _():
            ds = [pltpu.async_copy(xhi_hbm.at[colv.at[k]],
                                   gath.at[pl.ds(k * SUB, SUB)], gsem)
                  for k in range(NSUB)]
            for d in ds:
                d.wait()

        def scale(j, c2):
            v16 = valv[pl.ds((j >> 4) << 4, LANES)]
            vv = jnp.take_along_axis(
                v16, jnp.full((LANES,), j & (LANES - 1), _i32), axis=0)
            gath[j] = gath[j] * vv
            return c2

        lax.fori_loop(0, CH, scale, 0)

        sds = [pltpu.async_copy(gath.at[pl.ds(k * SUB, SUB)],
                                acc.at[rowv.at[k]], ssem, add=True)
               for k in range(NSUB)]
        for d in sds:
            d.wait()
        return carry

    lax.fori_loop(0, NCHUNK, chunk, 0)
    plsc.subcore_barrier()

    @pl.when(cid == 0)
    def _():
        pltpu.sync_copy(acc.at[pl.ds(sid * STRIPE, STRIPE)],
                        outlo_hbm.at[pl.ds(sid * STRIPE, STRIPE)])

        @pl.when(sid == 0)
        def _():
            pltpu.sync_copy(acc.at[pl.ds(TAIL_OFF, TAIL)],
                            outlo_hbm.at[pl.ds(TAIL_OFF, TAIL)])

    @pl.when(cid == 1)
    def _():
        pltpu.sync_copy(acc.at[pl.ds(sid * STRIPE, STRIPE)],
                        outhi_hbm.at[pl.ds(sid * STRIPE, STRIPE)])

        @pl.when(sid == 0)
        def _():
            pltpu.sync_copy(acc.at[pl.ds(TAIL_OFF, TAIL)],
                            outhi_hbm.at[pl.ds(TAIL_OFF, TAIL)])


@functools.cache
def _spmm_call():
    return pl.kernel(
        _spmm_body,
        out_type=[
            jax.ShapeDtypeStruct((N_ROWS, HALF), _f32),
            jax.ShapeDtypeStruct((N_ROWS, HALF), _f32),
        ],
        mesh=_mesh(),
        compiler_params=pltpu.CompilerParams(use_tc_tiling_on_sc=False, needs_layout_passes=False),
        scratch_types=[
            pltpu.VMEM((NSUB, SUB), _i32),
            pltpu.VMEM((NSUB, SUB), _i32),
            pltpu.VMEM((CH,), _f32),
            pltpu.VMEM((CH, HALF), _f32),
            pltpu.VMEM_SHARED((N_ROWS, HALF), _f32),
            pltpu.SemaphoreType.DMA,
            pltpu.SemaphoreType.DMA,
        ],
    )


def _prep_edges(indices, values):
    pad = E_PAD - N_EDGES
    dst = jnp.concatenate([indices[0], jnp.zeros((pad,), _i32)]).reshape(-1, SUB)
    src = jnp.concatenate([indices[1], jnp.zeros((pad,), _i32)]).reshape(-1, SUB)
    val = jnp.concatenate([values, jnp.zeros((pad,), _f32)])
    return dst, src, val


BPW = N_BATCH // (NC * NS)
LSUB = BPW // SUB


def _predict_body(ul_hbm, il_hbm, ulo_hbm, uhi_hbm, flo_hbm, fhi_hbm,
                  pv_hbm, pr_hbm, ulv, ilv, gul, guh, gfl, gfh, pvv, prv, sem):
    cid = lax.axis_index("c")
    sid = lax.axis_index("s")
    w = sid * NC + cid

    pltpu.sync_copy(ul_hbm.at[pl.ds(w * BPW, BPW)], ulv)
    pltpu.sync_copy(il_hbm.at[pl.ds(w * BPW, BPW)], ilv)

    ds = []
    for k in range(LSUB):
        ds.append(pltpu.async_copy(ulo_hbm.at[ulv.at[pl.ds(k * SUB, SUB)]],
                                   gul.at[pl.ds(k * SUB, SUB)], sem))
        ds.append(pltpu.async_copy(uhi_hbm.at[ulv.at[pl.ds(k * SUB, SUB)]],
                                   guh.at[pl.ds(k * SUB, SUB)], sem))
        ds.append(pltpu.async_copy(flo_hbm.at[ilv.at[pl.ds(k * SUB, SUB)]],
                                   gfl.at[pl.ds(k * SUB, SUB)], sem))
        ds.append(pltpu.async_copy(fhi_hbm.at[ilv.at[pl.ds(k * SUB, SUB)]],
                                   gfh.at[pl.ds(k * SUB, SUB)], sem))
    for d in ds:
        d.wait()

    lane = lax.iota(_i32, LANES)

    def group(g, c2):
        sums = jnp.zeros((LANES,), _f32)
        for i in range(LANES):
            r = g * LANES + i
            a = gul[r] * gfl[r] + guh[r] * gfh[r]
            for k in (1, 2, 4, 8):
                a = a + jnp.take_along_axis(a, lane ^ k, axis=0)
            sums = jnp.where(lane == i, a, sums)
        pvv[pl.ds(g * LANES, LANES)] = sums
        prv[pl.ds(g * LANES, LANES)] = 1.0 / (1.0 + jnp.exp(-sums))
        return c2

    lax.fori_loop(0, BPW // LANES, group, 0)

    pltpu.sync_copy(pvv, pv_hbm.at[pl.ds(w * BPW, BPW)])
    pltpu.sync_copy(prv, pr_hbm.at[pl.ds(w * BPW, BPW)])


@functools.cache
def _predict_call():
    return pl.kernel(
        _predict_body,
        out_type=[
            jax.ShapeDtypeStruct((N_BATCH,), _f32),
            jax.ShapeDtypeStruct((N_BATCH,), _f32),
        ],
        mesh=_mesh(),
        compiler_params=pltpu.CompilerParams(
            use_tc_tiling_on_sc=False, needs_layout_passes=False),
        scratch_types=[
        pltpu.VMEM((BPW,), _i32),
        pltpu.VMEM((BPW,), _i32),
        pltpu.VMEM((BPW, HALF), _f32),
        pltpu.VMEM((BPW, HALF), _f32),
        pltpu.VMEM((BPW, HALF), _f32),
        pltpu.VMEM((BPW, HALF), _f32),
            pltpu.VMEM((BPW,), _f32),
            pltpu.VMEM((BPW,), _f32),
            pltpu.SemaphoreType.DMA,
        ],
    )


def kernel(user_embedding, item_embedding, user_features, item_features,
           wr_w, wr_b, social_indices, social_values, ui_indices, ui_values,
           user_list, item_list):
    uf_lo, uf_hi = _fusion_halves(user_features, wr_w, wr_b, user_embedding)
    if_lo, if_hi = _fusion_halves(item_features, wr_w, wr_b, item_embedding)

    s_dst, s_src, s_val = _prep_edges(social_indices, social_values)
    u_dst, u_src, u_val = _prep_edges(ui_indices, ui_values)
    zeros = jnp.zeros((N_ROWS, HALF), _f32)

    ic_lo, ic_hi = _spmm_call()(u_dst, u_src, u_val, if_lo, if_hi, zeros, zeros)
    h1_lo, h1_hi = _spmm_call()(s_dst, s_src, s_val, uf_lo, uf_hi, zeros, zeros)
    lu_lo, lu_hi = _spmm_call()(s_dst, s_src, s_val, h1_lo, h1_hi, ic_lo, ic_hi)

    pv, pr = _predict_call()(user_list, item_list, lu_lo, lu_hi, if_lo, if_hi)
    return pv, pr


# speedup vs baseline: 9.6727x; 1.6990x over previous
import functools

import jax
import jax.numpy as jnp
from jax import lax
from jax.experimental import pallas as pl
from jax.experimental.pallas import tpu as pltpu
from jax.experimental.pallas import tpu_sc as plsc

N_ROWS = 100000
N_FEAT = 128
N_DIM = 32
HALF = 16
N_EDGES = 1600000
N_BATCH = 16384

NC = 2
NS = 16
LANES = 16

CH = 1024
SUB = 128
NSUB = CH // SUB
EPW = 100352
NCHUNK = EPW // CH
E_PAD = EPW * NS
NRW = EPW // SUB
STRIPE = 6248
TAIL_OFF = STRIPE * NS
TAIL = N_ROWS - TAIL_OFF

BR = 1000
GRID = N_ROWS // BR

_f32 = jnp.float32
_i32 = jnp.int32


def _mm_stats_body(f_ref, w_ref, z_ref, s1_ref, s2_ref, wcs_ref):
    i = pl.program_id(0)
    f = f_ref[...]
    z_ref[...] = jnp.dot(f, w_ref[...], preferred_element_type=_f32)

    @pl.when(i == 0)
    def _():
        s1_ref[...] = jnp.zeros((1, 1), _f32)
        s2_ref[...] = jnp.zeros((1, 1), _f32)
        wcs_ref[...] = jnp.sum(w_ref[...], axis=0, keepdims=True)

    s1_ref[...] += jnp.sum(f).reshape(1, 1)
    s2_ref[...] += jnp.sum(f * f).reshape(1, 1)


def _mm_stats(features, w):
    return pl.pallas_call(
        _mm_stats_body,
        grid=(GRID,),
        in_specs=[
            pl.BlockSpec((BR, N_FEAT), lambda i: (i, 0)),
            pl.BlockSpec((N_FEAT, N_DIM), lambda i: (0, 0)),
        ],
        out_specs=[
            pl.BlockSpec((BR, N_DIM), lambda i: (i, 0)),
            pl.BlockSpec((1, 1), lambda i: (0, 0)),
            pl.BlockSpec((1, 1), lambda i: (0, 0)),
            pl.BlockSpec((1, N_DIM), lambda i: (0, 0)),
        ],
        out_shape=[
            jax.ShapeDtypeStruct((N_ROWS, N_DIM), _f32),
            jax.ShapeDtypeStruct((1, 1), _f32),
            jax.ShapeDtypeStruct((1, 1), _f32),
            jax.ShapeDtypeStruct((1, N_DIM), _f32),
        ],
    )(features, w)


def _sig_stats_body(z_ref, a_ref, c_ref, s_ref, s1_ref, s2_ref):
    i = pl.program_id(0)
    z = z_ref[...]
    s = 1.0 / (1.0 + jnp.exp(-(z * a_ref[...] + c_ref[...])))
    s_ref[...] = s

    @pl.when(i == 0)
    def _():
        s1_ref[...] = jnp.zeros((1, 1), _f32)
        s2_ref[...] = jnp.zeros((1, 1), _f32)

    s1_ref[...] += jnp.sum(s).reshape(1, 1)
    s2_ref[...] += jnp.sum(s * s).reshape(1, 1)


def _sig_stats(z, a, c):
    return pl.pallas_call(
        _sig_stats_body,
        grid=(GRID,),
        in_specs=[
            pl.BlockSpec((BR, N_DIM), lambda i: (i, 0)),
            pl.BlockSpec((1, 1), lambda i: (0, 0)),
            pl.BlockSpec((1, N_DIM), lambda i: (0, 0)),
        ],
        out_specs=[
            pl.BlockSpec((BR, N_DIM), lambda i: (i, 0)),
            pl.BlockSpec((1, 1), lambda i: (0, 0)),
            pl.BlockSpec((1, 1), lambda i: (0, 0)),
        ],
        out_shape=[
            jax.ShapeDtypeStruct((N_ROWS, N_DIM), _f32),
            jax.ShapeDtypeStruct((1, 1), _f32),
            jax.ShapeDtypeStruct((1, 1), _f32),
        ],
    )(z, a, c)


def _fusion_body(s_ref, emb_ref, m_ref, is_ref, lo_ref, hi_ref):
    fus = (s_ref[...] - m_ref[...]) * is_ref[...] + emb_ref[...]
    lo_ref[...] = fus[:, :HALF]
    hi_ref[...] = fus[:, HALF:]


def _fusion(s, emb, m, inv_s):
    return pl.pallas_call(
        _fusion_body,
        grid=(GRID,),
        in_specs=[
            pl.BlockSpec((BR, N_DIM), lambda i: (i, 0)),
            pl.BlockSpec((BR, N_DIM), lambda i: (i, 0)),
            pl.BlockSpec((1, 1), lambda i: (0, 0)),
            pl.BlockSpec((1, 1), lambda i: (0, 0)),
        ],
        out_specs=[
            pl.BlockSpec((BR, HALF), lambda i: (i, 0)),
            pl.BlockSpec((BR, HALF), lambda i: (i, 0)),
        ],
        out_shape=[
            jax.ShapeDtypeStruct((N_ROWS, HALF), _f32),
            jax.ShapeDtypeStruct((N_ROWS, HALF), _f32),
        ],
    )(s, emb, m, inv_s)


def _fusion_halves(features, w, b, emb):
    z, s1, s2, wcs = _mm_stats(features, w)
    n = float(N_ROWS * N_FEAT)
    m = s1 / n
    inv_s = 1.0 / jnp.sqrt((s2 - n * m * m) / (n - 1.0))
    c = b.reshape(1, N_DIM) - (m * inv_s) * wcs
    s, t1, t2 = _sig_stats(z, inv_s, c)
    n2 = float(N_ROWS * N_DIM)
    m2 = t1 / n2
    inv_s2 = 1.0 / jnp.sqrt((t2 - n2 * m2 * m2) / (n2 - 1.0))
    return _fusion(s, emb, m2, inv_s2)


@functools.cache
def _mesh():
    return plsc.VectorSubcoreMesh(
        core_axis_name="c", subcore_axis_name="s",
        num_cores=NC, num_subcores=NS)


def _spmm_body(dst_hbm, src_hbm, val_hbm, xlo_hbm, xhi_hbm, ilo_hbm, ihi_hbm,
               outlo_hbm, outhi_hbm, colv, rowv, valv, gath, acc, gsem, ssem):
    cid = lax.axis_index("c")
    sid = lax.axis_index("s")

    @pl.when(cid == 0)
    def _():
        pltpu.sync_copy(ilo_hbm.at[pl.ds(sid * STRIPE, STRIPE)],
                        acc.at[pl.ds(sid * STRIPE, STRIPE)])

        @pl.when(sid == 0)
        def _():
            pltpu.sync_copy(ilo_hbm.at[pl.ds(TAIL_OFF, TAIL)],
                            acc.at[pl.ds(TAIL_OFF, TAIL)])

    @pl.when(cid == 1)
    def _():
        pltpu.sync_copy(ihi_hbm.at[pl.ds(sid * STRIPE, STRIPE)],
                        acc.at[pl.ds(sid * STRIPE, STRIPE)])

        @pl.when(sid == 0)
        def _():
            pltpu.sync_copy(ihi_hbm.at[pl.ds(TAIL_OFF, TAIL)],
                            acc.at[pl.ds(TAIL_OFF, TAIL)])

    plsc.subcore_barrier()

    def chunk(ci, carry):
        r0 = sid * NRW + ci * NSUB
        pltpu.sync_copy(src_hbm.at[pl.ds(r0, NSUB)], colv)
        pltpu.sync_copy(dst_hbm.at[pl.ds(r0, NSUB)], rowv)
        pltpu.sync_copy(val_hbm.at[pl.ds(sid * EPW + ci * CH, CH)], valv)

        @pl.when(cid == 0)
        def _():
            ds = [pltpu.async_copy(xlo_hbm.at[colv.at[k]],
                                   gath.at[pl.ds(k * SUB, SUB)], gsem)
                  for k in range(NSUB)]
            for d in ds:
                d.wait()

        @pl.when(cid == 1)
        def _():
            ds = [pltpu.async_copy(xhi_hbm.at[colv.at[k]],
                                   gath.at[pl.ds(k * SUB, SUB)], gsem)
                  for k in range(NSUB)]
            for d in ds:
                d.wait()

        def scale(g, c2):
            base = g * LANES
            v16 = valv[pl.ds(base, LANES)]
            for i in range(LANES):
                vv = jnp.take_along_axis(
                    v16, jnp.full((LANES,), i, _i32), axis=0)
                gath[base + i] = gath[base + i] * vv
            return c2

        lax.fori_loop(0, CH // LANES, scale, 0)

        sds = [pltpu.async_copy(gath.at[pl.ds(k * SUB, SUB)],
                                acc.at[rowv.at[k]], ssem, add=True)
               for k in range(NSUB)]
        for d in sds:
            d.wait()
        return carry

    lax.fori_loop(0, NCHUNK, chunk, 0)
    plsc.subcore_barrier()

    @pl.when(cid == 0)
    def _():
        pltpu.sync_copy(acc.at[pl.ds(sid * STRIPE, STRIPE)],
                        outlo_hbm.at[pl.ds(sid * STRIPE, STRIPE)])

        @pl.when(sid == 0)
        def _():
            pltpu.sync_copy(acc.at[pl.ds(TAIL_OFF, TAIL)],
                            outlo_hbm.at[pl.ds(TAIL_OFF, TAIL)])

    @pl.when(cid == 1)
    def _():
        pltpu.sync_copy(acc.at[pl.ds(sid * STRIPE, STRIPE)],
                        outhi_hbm.at[pl.ds(sid * STRIPE, STRIPE)])

        @pl.when(sid == 0)
        def _():
            pltpu.sync_copy(acc.at[pl.ds(TAIL_OFF, TAIL)],
                            outhi_hbm.at[pl.ds(TAIL_OFF, TAIL)])


@functools.cache
def _spmm_call():
    return pl.kernel(
        _spmm_body,
        out_type=[
            jax.ShapeDtypeStruct((N_ROWS, HALF), _f32),
            jax.ShapeDtypeStruct((N_ROWS, HALF), _f32),
        ],
        mesh=_mesh(),
        compiler_params=pltpu.CompilerParams(use_tc_tiling_on_sc=False, needs_layout_passes=False),
        scratch_types=[
            pltpu.VMEM((NSUB, SUB), _i32),
            pltpu.VMEM((NSUB, SUB), _i32),
            pltpu.VMEM((CH,), _f32),
            pltpu.VMEM((CH, HALF), _f32),
            pltpu.VMEM_SHARED((N_ROWS, HALF), _f32),
            pltpu.SemaphoreType.DMA,
            pltpu.SemaphoreType.DMA,
        ],
    )


def _prep_edges(indices, values):
    pad = E_PAD - N_EDGES
    dst = jnp.concatenate([indices[0], jnp.zeros((pad,), _i32)]).reshape(-1, SUB)
    src = jnp.concatenate([indices[1], jnp.zeros((pad,), _i32)]).reshape(-1, SUB)
    val = jnp.concatenate([values, jnp.zeros((pad,), _f32)])
    return dst, src, val


BPW = N_BATCH // (NC * NS)
LSUB = BPW // SUB


def _predict_body(ul_hbm, il_hbm, ulo_hbm, uhi_hbm, flo_hbm, fhi_hbm,
                  pv_hbm, pr_hbm, ulv, ilv, gul, guh, gfl, gfh, pvv, prv, sem):
    cid = lax.axis_index("c")
    sid = lax.axis_index("s")
    w = sid * NC + cid

    pltpu.sync_copy(ul_hbm.at[pl.ds(w * BPW, BPW)], ulv)
    pltpu.sync_copy(il_hbm.at[pl.ds(w * BPW, BPW)], ilv)

    ds = []
    for k in range(LSUB):
        ds.append(pltpu.async_copy(ulo_hbm.at[ulv.at[pl.ds(k * SUB, SUB)]],
                                   gul.at[pl.ds(k * SUB, SUB)], sem))
        ds.append(pltpu.async_copy(uhi_hbm.at[ulv.at[pl.ds(k * SUB, SUB)]],
                                   guh.at[pl.ds(k * SUB, SUB)], sem))
        ds.append(pltpu.async_copy(flo_hbm.at[ilv.at[pl.ds(k * SUB, SUB)]],
                                   gfl.at[pl.ds(k * SUB, SUB)], sem))
        ds.append(pltpu.async_copy(fhi_hbm.at[ilv.at[pl.ds(k * SUB, SUB)]],
                                   gfh.at[pl.ds(k * SUB, SUB)], sem))
    for d in ds:
        d.wait()

    lane = lax.iota(_i32, LANES)

    def group(g, c2):
        sums = jnp.zeros((LANES,), _f32)
        for i in range(LANES):
            r = g * LANES + i
            a = gul[r] * gfl[r] + guh[r] * gfh[r]
            for k in (1, 2, 4, 8):
                a = a + jnp.take_along_axis(a, lane ^ k, axis=0)
            sums = jnp.where(lane == i, a, sums)
        pvv[pl.ds(g * LANES, LANES)] = sums
        prv[pl.ds(g * LANES, LANES)] = 1.0 / (1.0 + jnp.exp(-sums))
        return c2

    lax.fori_loop(0, BPW // LANES, group, 0)

    pltpu.sync_copy(pvv, pv_hbm.at[pl.ds(w * BPW, BPW)])
    pltpu.sync_copy(prv, pr_hbm.at[pl.ds(w * BPW, BPW)])


@functools.cache
def _predict_call():
    return pl.kernel(
        _predict_body,
        out_type=[
            jax.ShapeDtypeStruct((N_BATCH,), _f32),
            jax.ShapeDtypeStruct((N_BATCH,), _f32),
        ],
        mesh=_mesh(),
        compiler_params=pltpu.CompilerParams(
            use_tc_tiling_on_sc=False, needs_layout_passes=False),
        scratch_types=[
        pltpu.VMEM((BPW,), _i32),
        pltpu.VMEM((BPW,), _i32),
        pltpu.VMEM((BPW, HALF), _f32),
        pltpu.VMEM((BPW, HALF), _f32),
        pltpu.VMEM((BPW, HALF), _f32),
        pltpu.VMEM((BPW, HALF), _f32),
            pltpu.VMEM((BPW,), _f32),
            pltpu.VMEM((BPW,), _f32),
            pltpu.SemaphoreType.DMA,
        ],
    )


def kernel(user_embedding, item_embedding, user_features, item_features,
           wr_w, wr_b, social_indices, social_values, ui_indices, ui_values,
           user_list, item_list):
    uf_lo, uf_hi = _fusion_halves(user_features, wr_w, wr_b, user_embedding)
    if_lo, if_hi = _fusion_halves(item_features, wr_w, wr_b, item_embedding)

    s_dst, s_src, s_val = _prep_edges(social_indices, social_values)
    u_dst, u_src, u_val = _prep_edges(ui_indices, ui_values)
    zeros = jnp.zeros((N_ROWS, HALF), _f32)

    ic_lo, ic_hi = _spmm_call()(u_dst, u_src, u_val, if_lo, if_hi, zeros, zeros)
    h1_lo, h1_hi = _spmm_call()(s_dst, s_src, s_val, uf_lo, uf_hi, zeros, zeros)
    lu_lo, lu_hi = _spmm_call()(s_dst, s_src, s_val, h1_lo, h1_hi, ic_lo, ic_hi)

    pv, pr = _predict_call()(user_list, item_list, lu_lo, lu_hi, if_lo, if_hi)
    return pv, pr


# speedup vs baseline: 11.6508x; 1.2045x over previous
import functools

import jax
import jax.numpy as jnp
from jax import lax
from jax.experimental import pallas as pl
from jax.experimental.pallas import tpu as pltpu
from jax.experimental.pallas import tpu_sc as plsc

N_ROWS = 100000
N_FEAT = 128
N_DIM = 32
HALF = 16
N_EDGES = 1600000
N_BATCH = 16384

NC = 2
NS = 16
LANES = 16

CH = 512
SUB = 128
NSUB = CH // SUB
NCHUNK = 198
EPW = NCHUNK * CH
HEAD = 6
TAILC = 6
NSTEADY = (NCHUNK - HEAD - TAILC) // 6
E_PAD = EPW * NS
NRW = EPW // SUB
STRIPE = 6248
TAIL_OFF = STRIPE * NS
TAIL = N_ROWS - TAIL_OFF

BR = 1000
GRID = N_ROWS // BR

_f32 = jnp.float32
_i32 = jnp.int32


def _mm_stats_body(f_ref, w_ref, z_ref, s1_ref, s2_ref, wcs_ref):
    i = pl.program_id(0)
    f = f_ref[...]
    z_ref[...] = jnp.dot(f, w_ref[...], preferred_element_type=_f32)

    @pl.when(i == 0)
    def _():
        s1_ref[...] = jnp.zeros((1, 1), _f32)
        s2_ref[...] = jnp.zeros((1, 1), _f32)
        wcs_ref[...] = jnp.sum(w_ref[...], axis=0, keepdims=True)

    s1_ref[...] += jnp.sum(f).reshape(1, 1)
    s2_ref[...] += jnp.sum(f * f).reshape(1, 1)


def _mm_stats(features, w):
    return pl.pallas_call(
        _mm_stats_body,
        grid=(GRID,),
        in_specs=[
            pl.BlockSpec((BR, N_FEAT), lambda i: (i, 0)),
            pl.BlockSpec((N_FEAT, N_DIM), lambda i: (0, 0)),
        ],
        out_specs=[
            pl.BlockSpec((BR, N_DIM), lambda i: (i, 0)),
            pl.BlockSpec((1, 1), lambda i: (0, 0)),
            pl.BlockSpec((1, 1), lambda i: (0, 0)),
            pl.BlockSpec((1, N_DIM), lambda i: (0, 0)),
        ],
        out_shape=[
            jax.ShapeDtypeStruct((N_ROWS, N_DIM), _f32),
            jax.ShapeDtypeStruct((1, 1), _f32),
            jax.ShapeDtypeStruct((1, 1), _f32),
            jax.ShapeDtypeStruct((1, N_DIM), _f32),
        ],
    )(features, w)


def _sig_stats_body(z_ref, a_ref, c_ref, s_ref, s1_ref, s2_ref):
    i = pl.program_id(0)
    z = z_ref[...]
    s = 1.0 / (1.0 + jnp.exp(-(z * a_ref[...] + c_ref[...])))
    s_ref[...] = s

    @pl.when(i == 0)
    def _():
        s1_ref[...] = jnp.zeros((1, 1), _f32)
        s2_ref[...] = jnp.zeros((1, 1), _f32)

    s1_ref[...] += jnp.sum(s).reshape(1, 1)
    s2_ref[...] += jnp.sum(s * s).reshape(1, 1)


def _sig_stats(z, a, c):
    return pl.pallas_call(
        _sig_stats_body,
        grid=(GRID,),
        in_specs=[
            pl.BlockSpec((BR, N_DIM), lambda i: (i, 0)),
            pl.BlockSpec((1, 1), lambda i: (0, 0)),
            pl.BlockSpec((1, N_DIM), lambda i: (0, 0)),
        ],
        out_specs=[
            pl.BlockSpec((BR, N_DIM), lambda i: (i, 0)),
            pl.BlockSpec((1, 1), lambda i: (0, 0)),
            pl.BlockSpec((1, 1), lambda i: (0, 0)),
        ],
        out_shape=[
            jax.ShapeDtypeStruct((N_ROWS, N_DIM), _f32),
            jax.ShapeDtypeStruct((1, 1), _f32),
            jax.ShapeDtypeStruct((1, 1), _f32),
        ],
    )(z, a, c)


def _fusion_body(s_ref, emb_ref, m_ref, is_ref, lo_ref, hi_ref):
    fus = (s_ref[...] - m_ref[...]) * is_ref[...] + emb_ref[...]
    lo_ref[...] = fus[:, :HALF]
    hi_ref[...] = fus[:, HALF:]


def _fusion(s, emb, m, inv_s):
    return pl.pallas_call(
        _fusion_body,
        grid=(GRID,),
        in_specs=[
            pl.BlockSpec((BR, N_DIM), lambda i: (i, 0)),
            pl.BlockSpec((BR, N_DIM), lambda i: (i, 0)),
            pl.BlockSpec((1, 1), lambda i: (0, 0)),
            pl.BlockSpec((1, 1), lambda i: (0, 0)),
        ],
        out_specs=[
            pl.BlockSpec((BR, HALF), lambda i: (i, 0)),
            pl.BlockSpec((BR, HALF), lambda i: (i, 0)),
        ],
        out_shape=[
            jax.ShapeDtypeStruct((N_ROWS, HALF), _f32),
            jax.ShapeDtypeStruct((N_ROWS, HALF), _f32),
        ],
    )(s, emb, m, inv_s)


def _fusion_halves(features, w, b, emb):
    z, s1, s2, wcs = _mm_stats(features, w)
    n = float(N_ROWS * N_FEAT)
    m = s1 / n
    inv_s = 1.0 / jnp.sqrt((s2 - n * m * m) / (n - 1.0))
    c = b.reshape(1, N_DIM) - (m * inv_s) * wcs
    s, t1, t2 = _sig_stats(z, inv_s, c)
    n2 = float(N_ROWS * N_DIM)
    m2 = t1 / n2
    inv_s2 = 1.0 / jnp.sqrt((t2 - n2 * m2 * m2) / (n2 - 1.0))
    return _fusion(s, emb, m2, inv_s2)


@functools.cache
def _mesh():
    return plsc.VectorSubcoreMesh(
        core_axis_name="c", subcore_axis_name="s",
        num_cores=NC, num_subcores=NS)


def _spmm_body(dst_hbm, src_hbm, val_hbm, xlo_hbm, xhi_hbm, ilo_hbm, ihi_hbm,
               outlo_hbm, outhi_hbm,
               colv0, rowv0, valv0, colv1, rowv1, valv1, colv2, rowv2, valv2,
               gath0, gath1, acc,
               esem0, esem1, esem2, gsem0, gsem1, ssem0, ssem1):
    cid = lax.axis_index("c")
    sid = lax.axis_index("s")
    EB = ((colv0, rowv0, valv0, esem0),
          (colv1, rowv1, valv1, esem1),
          (colv2, rowv2, valv2, esem2))
    GB = ((gath0, gsem0, ssem0), (gath1, gsem1, ssem1))

    @pl.when(cid == 0)
    def _():
        pltpu.sync_copy(ilo_hbm.at[pl.ds(sid * STRIPE, STRIPE)],
                        acc.at[pl.ds(sid * STRIPE, STRIPE)])

        @pl.when(sid == 0)
        def _():
            pltpu.sync_copy(ilo_hbm.at[pl.ds(TAIL_OFF, TAIL)],
                            acc.at[pl.ds(TAIL_OFF, TAIL)])

    @pl.when(cid == 1)
    def _():
        pltpu.sync_copy(ihi_hbm.at[pl.ds(sid * STRIPE, STRIPE)],
                        acc.at[pl.ds(sid * STRIPE, STRIPE)])

        @pl.when(sid == 0)
        def _():
            pltpu.sync_copy(ihi_hbm.at[pl.ds(TAIL_OFF, TAIL)],
                            acc.at[pl.ds(TAIL_OFF, TAIL)])


    def fire_edges(ci, eb):
        colv, rowv, valv, esem = EB[eb]
        r0 = sid * NRW + ci * NSUB
        pltpu.async_copy(src_hbm.at[pl.ds(r0, NSUB)], colv, esem)
        pltpu.async_copy(dst_hbm.at[pl.ds(r0, NSUB)], rowv, esem)
        pltpu.async_copy(val_hbm.at[pl.ds(sid * EPW + ci * CH, CH)], valv,
                         esem)

    def wait_edges(ci, eb):
        colv, rowv, valv, esem = EB[eb]
        r0 = sid * NRW + ci * NSUB
        pltpu.make_async_copy(src_hbm.at[pl.ds(r0, NSUB)], colv, esem).wait()
        pltpu.make_async_copy(dst_hbm.at[pl.ds(r0, NSUB)], rowv, esem).wait()
        pltpu.make_async_copy(val_hbm.at[pl.ds(sid * EPW + ci * CH, CH)],
                              valv, esem).wait()

    def fire_gathers(eb, gb):
        colv = EB[eb][0]
        gath, gsem, _ = GB[gb]

        @pl.when(cid == 0)
        def _():
            for k in range(NSUB):
                pltpu.async_copy(xlo_hbm.at[colv.at[k]],
                                 gath.at[pl.ds(k * SUB, SUB)], gsem)

        @pl.when(cid == 1)
        def _():
            for k in range(NSUB):
                pltpu.async_copy(xhi_hbm.at[colv.at[k]],
                                 gath.at[pl.ds(k * SUB, SUB)], gsem)

    def wait_gathers(eb, gb):
        colv = EB[eb][0]
        gath, gsem, _ = GB[gb]
        for k in range(NSUB):
            pltpu.make_async_copy(xlo_hbm.at[colv.at[k]],
                                  gath.at[pl.ds(k * SUB, SUB)], gsem).wait()

    def scale(eb, gb):
        valv = EB[eb][2]
        gath = GB[gb][0]

        def body(g, c2):
            base = g * LANES
            v16 = valv[pl.ds(base, LANES)]
            for i in range(LANES):
                vv = jnp.take_along_axis(
                    v16, jnp.full((LANES,), i, _i32), axis=0)
                gath[base + i] = gath[base + i] * vv
            return c2

        lax.fori_loop(0, CH // LANES, body, 0)

    def fire_scatters(eb, gb):
        rowv = EB[eb][1]
        gath, _, ssem = GB[gb]
        for k in range(NSUB):
            pltpu.async_copy(gath.at[pl.ds(k * SUB, SUB)],
                             acc.at[rowv.at[k]], ssem, add=True)

    def wait_scatters(eb, gb):
        rowv = EB[eb][1]
        gath, _, ssem = GB[gb]
        for k in range(NSUB):
            pltpu.make_async_copy(gath.at[pl.ds(k * SUB, SUB)],
                                  acc.at[rowv.at[k]], ssem).wait()

    def step(ci, j, first=False, last_minus1=False, last=False):
        eb, gb = j % 3, j % 2
        wait_gathers(eb, gb)
        scale(eb, gb)
        fire_scatters(eb, gb)
        if not last:
            wait_edges(ci + 1, (eb + 1) % 3)
        if not first:
            wait_scatters((eb + 2) % 3, 1 - gb)
        if not last:
            fire_gathers((eb + 1) % 3, 1 - gb)
        if not (last or last_minus1):
            fire_edges(ci + 2, (eb + 2) % 3)

    fire_edges(0, 0)
    fire_edges(1, 1)
    wait_edges(0, 0)
    fire_gathers(0, 0)
    plsc.subcore_barrier()

    for j in range(HEAD):
        step(j, j, first=(j == 0))

    def six(p, c2):
        ci0 = HEAD + p * 6
        for j in range(6):
            step(ci0 + j, j)
        return c2

    lax.fori_loop(0, NSTEADY, six, 0)

    base = NCHUNK - TAILC
    for j in range(TAILC):
        step(base + j, j, last_minus1=(j == TAILC - 2), last=(j == TAILC - 1))

    wait_scatters((NCHUNK - 1) % 3, (NCHUNK - 1) % 2)
    plsc.subcore_barrier()

    @pl.when(cid == 0)
    def _():
        pltpu.sync_copy(acc.at[pl.ds(sid * STRIPE, STRIPE)],
                        outlo_hbm.at[pl.ds(sid * STRIPE, STRIPE)])

        @pl.when(sid == 0)
        def _():
            pltpu.sync_copy(acc.at[pl.ds(TAIL_OFF, TAIL)],
                            outlo_hbm.at[pl.ds(TAIL_OFF, TAIL)])

    @pl.when(cid == 1)
    def _():
        pltpu.sync_copy(acc.at[pl.ds(sid * STRIPE, STRIPE)],
                        outhi_hbm.at[pl.ds(sid * STRIPE, STRIPE)])

        @pl.when(sid == 0)
        def _():
            pltpu.sync_copy(acc.at[pl.ds(TAIL_OFF, TAIL)],
                            outhi_hbm.at[pl.ds(TAIL_OFF, TAIL)])


@functools.cache
def _spmm_call():
    return pl.kernel(
        _spmm_body,
        out_type=[
            jax.ShapeDtypeStruct((N_ROWS, HALF), _f32),
            jax.ShapeDtypeStruct((N_ROWS, HALF), _f32),
        ],
        mesh=_mesh(),
        compiler_params=pltpu.CompilerParams(use_tc_tiling_on_sc=False, needs_layout_passes=False),
        scratch_types=(
            [pltpu.VMEM((NSUB, SUB), _i32),
             pltpu.VMEM((NSUB, SUB), _i32),
             pltpu.VMEM((CH,), _f32)] * 3 +
            [pltpu.VMEM((CH, HALF), _f32),
             pltpu.VMEM((CH, HALF), _f32),
             pltpu.VMEM_SHARED((N_ROWS, HALF), _f32)] +
            [pltpu.SemaphoreType.DMA] * 7
        ),
    )


def _prep_edges(indices, values):
    pad = E_PAD - N_EDGES
    dst = jnp.concatenate([indices[0], jnp.zeros((pad,), _i32)]).reshape(-1, SUB)
    src = jnp.concatenate([indices[1], jnp.zeros((pad,), _i32)]).reshape(-1, SUB)
    val = jnp.concatenate([values, jnp.zeros((pad,), _f32)])
    return dst, src, val


BPW = N_BATCH // (NC * NS)
LSUB = BPW // SUB


def _predict_body(ul_hbm, il_hbm, ulo_hbm, uhi_hbm, flo_hbm, fhi_hbm,
                  pv_hbm, pr_hbm, ulv, ilv, gul, guh, gfl, gfh, pvv, prv, sem):
    cid = lax.axis_index("c")
    sid = lax.axis_index("s")
    w = sid * NC + cid

    pltpu.sync_copy(ul_hbm.at[pl.ds(w * BPW, BPW)], ulv)
    pltpu.sync_copy(il_hbm.at[pl.ds(w * BPW, BPW)], ilv)

    ds = []
    for k in range(LSUB):
        ds.append(pltpu.async_copy(ulo_hbm.at[ulv.at[pl.ds(k * SUB, SUB)]],
                                   gul.at[pl.ds(k * SUB, SUB)], sem))
        ds.append(pltpu.async_copy(uhi_hbm.at[ulv.at[pl.ds(k * SUB, SUB)]],
                                   guh.at[pl.ds(k * SUB, SUB)], sem))
        ds.append(pltpu.async_copy(flo_hbm.at[ilv.at[pl.ds(k * SUB, SUB)]],
                                   gfl.at[pl.ds(k * SUB, SUB)], sem))
        ds.append(pltpu.async_copy(fhi_hbm.at[ilv.at[pl.ds(k * SUB, SUB)]],
                                   gfh.at[pl.ds(k * SUB, SUB)], sem))
    for d in ds:
        d.wait()

    lane = lax.iota(_i32, LANES)

    def group(g, c2):
        sums = jnp.zeros((LANES,), _f32)
        for i in range(LANES):
            r = g * LANES + i
            a = gul[r] * gfl[r] + guh[r] * gfh[r]
            for k in (1, 2, 4, 8):
                a = a + jnp.take_along_axis(a, lane ^ k, axis=0)
            sums = jnp.where(lane == i, a, sums)
        pvv[pl.ds(g * LANES, LANES)] = sums
        prv[pl.ds(g * LANES, LANES)] = 1.0 / (1.0 + jnp.exp(-sums))
        return c2

    lax.fori_loop(0, BPW // LANES, group, 0)

    pltpu.sync_copy(pvv, pv_hbm.at[pl.ds(w * BPW, BPW)])
    pltpu.sync_copy(prv, pr_hbm.at[pl.ds(w * BPW, BPW)])


@functools.cache
def _predict_call():
    return pl.kernel(
        _predict_body,
        out_type=[
            jax.ShapeDtypeStruct((N_BATCH,), _f32),
            jax.ShapeDtypeStruct((N_BATCH,), _f32),
        ],
        mesh=_mesh(),
        compiler_params=pltpu.CompilerParams(
            use_tc_tiling_on_sc=False, needs_layout_passes=False),
        scratch_types=[
        pltpu.VMEM((BPW,), _i32),
        pltpu.VMEM((BPW,), _i32),
        pltpu.VMEM((BPW, HALF), _f32),
        pltpu.VMEM((BPW, HALF), _f32),
        pltpu.VMEM((BPW, HALF), _f32),
        pltpu.VMEM((BPW, HALF), _f32),
            pltpu.VMEM((BPW,), _f32),
            pltpu.VMEM((BPW,), _f32),
            pltpu.SemaphoreType.DMA,
        ],
    )


def kernel(user_embedding, item_embedding, user_features, item_features,
           wr_w, wr_b, social_indices, social_values, ui_indices, ui_values,
           user_list, item_list):
    uf_lo, uf_hi = _fusion_halves(user_features, wr_w, wr_b, user_embedding)
    if_lo, if_hi = _fusion_halves(item_features, wr_w, wr_b, item_embedding)

    s_dst, s_src, s_val = _prep_edges(social_indices, social_values)
    u_dst, u_src, u_val = _prep_edges(ui_indices, ui_values)
    zeros = jnp.zeros((N_ROWS, HALF), _f32)

    ic_lo, ic_hi = _spmm_call()(u_dst, u_src, u_val, if_lo, if_hi, zeros, zeros)
    h1_lo, h1_hi = _spmm_call()(s_dst, s_src, s_val, uf_lo, uf_hi, zeros, zeros)
    lu_lo, lu_hi = _spmm_call()(s_dst, s_src, s_val, h1_lo, h1_hi, ic_lo, ic_hi)

    pv, pr = _predict_call()(user_list, item_list, lu_lo, lu_hi, if_lo, if_hi)
    return pv, pr
